# Initial kernel scaffold; baseline (speedup 1.0000x reference)
import jax
import jax.numpy as jnp
from jax.experimental import pallas as pl


def kernel(x):
    raise NotImplementedError("write your pallas kernel here")


# speedup vs baseline: 7.8826x; 7.8826x over previous
import jax
import jax.numpy as jnp
from jax.experimental import pallas as pl
from jax.experimental.pallas import tpu as pltpu

_K = 64
_FACTOR = 6.26


def _search_ge(keys, valid, k, nbits, rows):
    def step(i, t):
        cand = t | (1 << (nbits - 1 - i))
        hit = valid & (keys >= cand)
        cnt = jnp.sum(hit.astype(jnp.int32), axis=1, keepdims=True)
        return jnp.where(cnt >= k, cand, t)
    return jax.lax.fori_loop(0, nbits, step,
                             jnp.zeros((rows, 1), jnp.int32))


def _side_mask(bits, rcol, rows):
    ones = bits >= 0
    t = _search_ge(bits, ones, _K, 31, rows)
    gt = bits > t
    cnt_gt = jnp.sum(gt.astype(jnp.int32), axis=1, keepdims=True)
    needed = _K - cnt_gt
    tie = bits == t
    t2 = _search_ge(rcol, tie, needed, 13, rows)
    return gt | (tie & (rcol >= t2))


def _body(x_ref, o_ref):
    x = x_ref[...]
    p = jnp.maximum(x, 0.0)
    n = jnp.maximum(-x, 0.0)
    pb = jax.lax.bitcast_convert_type(p, jnp.int32)
    nb = jax.lax.bitcast_convert_type(n, jnp.int32)
    rows, cols = x.shape
    rcol = jax.lax.broadcasted_iota(jnp.int32, (rows, cols), 1)
    rcol = (cols - 1) - rcol

    mp = _side_mask(pb, rcol, rows)
    mn = _side_mask(nb, rcol, rows)
    p_tmp = _FACTOR * jnp.sum(jnp.where(mp, 0.0, p), axis=1, keepdims=True)
    n_tmp = _FACTOR * jnp.sum(jnp.where(mn, 0.0, n), axis=1, keepdims=True)
    o_ref[...] = (jnp.where(mp, p + p_tmp, 0.0)
                  - jnp.where(mn, n + n_tmp, 0.0))


def kernel(x):
    rows, cols = x.shape
    blk = 16
    return pl.pallas_call(
        _body,
        grid=(rows // blk,),
        in_specs=[pl.BlockSpec((blk, cols), lambda i: (i, 0))],
        out_specs=pl.BlockSpec((blk, cols), lambda i: (i, 0)),
        out_shape=jax.ShapeDtypeStruct((rows, cols), x.dtype),
    )(x)


# speedup vs baseline: 7.9489x; 1.0084x over previous
import functools
import jax
import jax.numpy as jnp
from jax import lax
from jax.experimental import pallas as pl
from jax.experimental.pallas import tpu as pltpu
from jax.experimental.pallas import tpu_sc as plsc

_K = 64
_FACTOR = 6.26
_NC, _NS, _L = 2, 16, 16
_ROWS, _COLS = 128, 8192
_RPW = _ROWS // (_NC * _NS)
_NV = _COLS // _L


def _f32(bits):
    return plsc.bitcast(bits, jnp.float32)


def _select(ck, n0, sum_all):
    lanes = lax.iota(jnp.int32, _L)

    def count_pass(bit, n):
        nv = (n + _L - 1) // _L

        def body(i, carry):
            cnt, s = carry
            v = ck[pl.ds(i * _L, _L)]
            valid = (i * _L + lanes) < n
            hi = valid & (((v >> bit) & 1) == 1)
            cnt = cnt + jnp.sum(hi.astype(jnp.int32))
            s = s + jnp.sum(jnp.where(hi, _f32(v), 0.0))
            return cnt, s

        return lax.fori_loop(0, nv, body, (jnp.int32(0), jnp.float32(0.0)))

    def compact_pass(bit, n, want_hi):
        nv = (n + _L - 1) // _L

        def body(i, wp):
            v = ck[pl.ds(i * _L, _L)]
            valid = (i * _L + lanes) < n
            hi = ((v >> bit) & 1) == 1
            keep = valid & (hi == want_hi)
            plsc.store_compressed(ck.at[pl.ds(wp, _L)], v, mask=keep)
            return wp + jnp.sum(keep.astype(jnp.int32))

        return lax.fori_loop(0, nv, body, jnp.int32(0))

    def cond(state):
        bit, k, n, t, s_sel, s_cand = state
        return (bit >= 0) & (k < n)

    def step(state):
        bit, k, n, t, s_sel, s_cand = state
        cnt, s_hi = count_pass(bit, n)
        take_hi = cnt >= k
        new_n = compact_pass(bit, n, take_hi)
        t = jnp.where(take_hi, t | (1 << bit), t)
        k = jnp.where(take_hi, k, k - cnt)
        s_sel = jnp.where(take_hi, s_sel, s_sel + s_hi)
        s_cand = jnp.where(take_hi, s_hi, s_cand - s_hi)
        return bit - 1, k, new_n, t, s_sel, s_cand

    init = (jnp.int32(30), jnp.int32(_K), n0, jnp.int32(0),
            jnp.float32(0.0), sum_all)
    bit, k, n, t, s_sel, s_cand = lax.while_loop(cond, step, init)
    val_t = jnp.sum(jnp.where(lanes == 0,
                              _f32(jnp.full((_L,), t, jnp.int32)), 0.0))
    s_sel = s_sel + jnp.where(k == n, s_cand, k.astype(jnp.float32) * val_t)
    return t, k, s_sel


def _row_compute(xrow, orow, ckp, ckn):
    lanes = lax.iota(jnp.int32, _L)
    mask7f = jnp.full((_L,), 0x7FFFFFFF, jnp.int32)

    def p0(i, carry):
        sp, sn = carry
        v = xrow[pl.ds(i * _L, _L)]
        p = jnp.maximum(v, 0.0)
        nn = jnp.maximum(-v, 0.0)
        ckp[pl.ds(i * _L, _L)] = plsc.bitcast(p, jnp.int32) & mask7f
        ckn[pl.ds(i * _L, _L)] = plsc.bitcast(nn, jnp.int32) & mask7f
        return sp + jnp.sum(p), sn + jnp.sum(nn)

    sum_p, sum_n = lax.fori_loop(0, _NV, p0,
                                 (jnp.float32(0.0), jnp.float32(0.0)))

    tp, krp, ssp = _select(ckp, jnp.int32(_COLS), sum_p)
    tn, krn, ssn = _select(ckn, jnp.int32(_COLS), sum_n)

    p_tmp = _FACTOR * (sum_p - ssp)
    n_tmp = _FACTOR * (sum_n - ssn)

    def out_body(i, carry):
        tcp, tcn = carry
        v = xrow[pl.ds(i * _L, _L)]
        p = jnp.maximum(v, 0.0)
        nn = jnp.maximum(-v, 0.0)
        pk = plsc.bitcast(p, jnp.int32) & mask7f
        nk = plsc.bitcast(nn, jnp.int32) & mask7f

        tie_p = (pk == tp).astype(jnp.int32)
        rank_p = tcp + plsc.cumsum(tie_p)
        sel_p = (pk > tp) | ((tie_p == 1) & (rank_p <= krp))
        tcp = tcp + jnp.sum(tie_p)

        tie_n = (nk == tn).astype(jnp.int32)
        rank_n = tcn + plsc.cumsum(tie_n)
        sel_n = (nk > tn) | ((tie_n == 1) & (rank_n <= krn))
        tcn = tcn + jnp.sum(tie_n)

        out = (jnp.where(sel_p, p + p_tmp, 0.0)
               - jnp.where(sel_n, nn + n_tmp, 0.0))
        orow[pl.ds(i * _L, _L)] = out
        return tcp, tcn

    lax.fori_loop(0, _NV, out_body, (jnp.int32(0), jnp.int32(0)))


def _body(x_hbm, o_hbm, xrow, orow, ckp, ckn):
    wid = lax.axis_index("s") * _NC + lax.axis_index("c")

    def per_row(r, carry):
        row = wid * _RPW + r
        pltpu.sync_copy(x_hbm.at[row], xrow)
        _row_compute(xrow, orow, ckp, ckn)
        pltpu.sync_copy(orow, o_hbm.at[row])
        return carry

    lax.fori_loop(0, _RPW, per_row, jnp.int32(0))


def sc_kernel(x):
    mesh = plsc.VectorSubcoreMesh(core_axis_name="c", subcore_axis_name="s",
                                  num_cores=_NC, num_subcores=_NS)
    f = pl.kernel(
        _body,
        out_type=jax.ShapeDtypeStruct((_ROWS, _COLS), jnp.float32),
        mesh=mesh,
        compiler_params=pltpu.CompilerParams(needs_layout_passes=False),
        scratch_types=[
            pltpu.VMEM((_COLS,), jnp.float32),
            pltpu.VMEM((_COLS,), jnp.float32),
            pltpu.VMEM((_COLS + 2 * _L,), jnp.int32),
            pltpu.VMEM((_COLS + 2 * _L,), jnp.int32),
        ],
    )
    return f(x)


def kernel(x):
    return sc_kernel(x)
